# baseline (device time: 50820 ns/iter reference)
import jax
import jax.numpy as jnp
from jax import lax
from jax.experimental import pallas as pl
from jax.experimental.pallas import tpu as pltpu

N_DEV = 16
ROUNDS = 4


def _gelu(y):
    c = 0.7978845608028654
    return 0.5 * y * (1.0 + jnp.tanh(c * (y + 0.044715 * y * y * y)))


def kernel(x, w_mat):
    k_global, k_shard = x.shape
    n = w_mat.shape[1]
    m_per = k_global // N_DEV
    assert k_shard == m_per

    def body(x_ref, w_ref, out_ref, xsend_ref, xcomm_ref,
             send_sems, recv_sems, bar_sems):
        my = lax.axis_index("i")

        xsend_ref[:, :] = x_ref[:, :].astype(jnp.bfloat16)

        sends = []

        def send_block(s):
            dest = lax.rem(my - s + N_DEV, N_DEV)
            rdma = pltpu.make_async_remote_copy(
                src_ref=xsend_ref.at[pl.ds(dest * m_per, m_per)],
                dst_ref=xcomm_ref.at[my],
                send_sem=send_sems.at[s],
                recv_sem=recv_sems.at[my],
                device_id=(dest,),
                device_id_type=pl.DeviceIdType.MESH,
            )
            rdma.start()
            sends.append(rdma)

        barrier_sem = pltpu.get_barrier_semaphore()
        for r in range(ROUNDS):
            off = 1 << r
            round_sem = barrier_sem if r == 0 else bar_sems.at[r]
            pl.semaphore_signal(
                round_sem, inc=1,
                device_id=(lax.rem(my + off, N_DEV),),
                device_id_type=pl.DeviceIdType.MESH,
            )
            if r == 0:
                acc = jnp.dot(
                    x_ref[pl.ds(my * m_per, m_per), :],
                    w_ref[pl.ds(my * m_per, m_per), :],
                    preferred_element_type=jnp.float32,
                )
            pl.semaphore_wait(round_sem, 1)
            for s in range(off, min(off << 1, N_DEV)):
                send_block(s)

        for s in range(1, N_DEV):
            src = lax.rem(my + s, N_DEV)
            recv = pltpu.make_async_remote_copy(
                src_ref=xsend_ref.at[pl.ds(0, m_per)],
                dst_ref=xcomm_ref.at[src],
                send_sem=send_sems.at[0],
                recv_sem=recv_sems.at[src],
                device_id=(src,),
                device_id_type=pl.DeviceIdType.MESH,
            )
            recv.wait_recv()
            acc = acc + jnp.dot(
                xcomm_ref[src].astype(jnp.float32),
                w_ref[pl.ds(src * m_per, m_per), :],
                preferred_element_type=jnp.float32,
            )

        out_ref[:, :] = _gelu(acc)

        for rdma in sends:
            rdma.wait_send()

    return pl.pallas_call(
        body,
        out_shape=jax.ShapeDtypeStruct((m_per, n), jnp.float32),
        in_specs=[
            pl.BlockSpec(memory_space=pltpu.VMEM),
            pl.BlockSpec(memory_space=pltpu.VMEM),
        ],
        out_specs=pl.BlockSpec(memory_space=pltpu.VMEM),
        scratch_shapes=[
            pltpu.VMEM((k_global, m_per), jnp.bfloat16),
            pltpu.VMEM((N_DEV, m_per, m_per), jnp.bfloat16),
            pltpu.SemaphoreType.DMA((N_DEV,)),
            pltpu.SemaphoreType.DMA((N_DEV,)),
            pltpu.SemaphoreType.REGULAR((ROUNDS,)),
        ],
        compiler_params=pltpu.CompilerParams(
            vmem_limit_bytes=100 * 1024 * 1024,
            collective_id=0,
        ),
    )(x, w_mat)


# device time: 50411 ns/iter; 1.0081x vs baseline; 1.0081x over previous
import jax
import jax.numpy as jnp
from jax import lax
from jax.experimental import pallas as pl
from jax.experimental.pallas import tpu as pltpu

N_DEV = 16


def _gelu(y):
    c = 0.7978845608028654
    return 0.5 * y * (1.0 + jnp.tanh(c * (y + 0.044715 * y * y * y)))


def kernel(x, w_mat):
    k_global, k_shard = x.shape
    n = w_mat.shape[1]
    m_per = k_global // N_DEV
    assert k_shard == m_per

    def body(x_ref, w_ref, out_ref, xsend_ref, xcomm_ref,
             send_sems, recv_sems, credit_sems):
        my = lax.axis_index("i")

        barrier_sem = pltpu.get_barrier_semaphore()
        pl.semaphore_signal(
            barrier_sem, inc=1,
            device_id=(my,), device_id_type=pl.DeviceIdType.MESH,
        )

        xsend_ref[:, :] = x_ref[:, :].astype(jnp.bfloat16)

        for s in range(1, N_DEV):
            pl.semaphore_signal(
                credit_sems.at[my], inc=1,
                device_id=(lax.rem(my + s, N_DEV),),
                device_id_type=pl.DeviceIdType.MESH,
            )
        pl.semaphore_wait(barrier_sem, 1)

        acc = jnp.dot(
            x_ref[pl.ds(my * m_per, m_per), :],
            w_ref[pl.ds(my * m_per, m_per), :],
            preferred_element_type=jnp.float32,
        )

        sends = []
        for s in range(1, N_DEV):
            dest = lax.rem(my - s + N_DEV, N_DEV)
            pl.semaphore_wait(credit_sems.at[dest], 1)
            rdma = pltpu.make_async_remote_copy(
                src_ref=xsend_ref.at[pl.ds(dest * m_per, m_per)],
                dst_ref=xcomm_ref.at[my],
                send_sem=send_sems.at[s],
                recv_sem=recv_sems.at[my],
                device_id=(dest,),
                device_id_type=pl.DeviceIdType.MESH,
            )
            rdma.start()
            sends.append(rdma)

        for s in range(1, N_DEV):
            src = lax.rem(my + s, N_DEV)
            recv = pltpu.make_async_remote_copy(
                src_ref=xsend_ref.at[pl.ds(0, m_per)],
                dst_ref=xcomm_ref.at[src],
                send_sem=send_sems.at[0],
                recv_sem=recv_sems.at[src],
                device_id=(src,),
                device_id_type=pl.DeviceIdType.MESH,
            )
            recv.wait_recv()
            acc = acc + jnp.dot(
                xcomm_ref[src].astype(jnp.float32),
                w_ref[pl.ds(src * m_per, m_per), :],
                preferred_element_type=jnp.float32,
            )

        out_ref[:, :] = _gelu(acc)

        for rdma in sends:
            rdma.wait_send()

    return pl.pallas_call(
        body,
        out_shape=jax.ShapeDtypeStruct((m_per, n), jnp.float32),
        in_specs=[
            pl.BlockSpec(memory_space=pltpu.VMEM),
            pl.BlockSpec(memory_space=pltpu.VMEM),
        ],
        out_specs=pl.BlockSpec(memory_space=pltpu.VMEM),
        scratch_shapes=[
            pltpu.VMEM((k_global, m_per), jnp.bfloat16),
            pltpu.VMEM((N_DEV, m_per, m_per), jnp.bfloat16),
            pltpu.SemaphoreType.DMA((N_DEV,)),
            pltpu.SemaphoreType.DMA((N_DEV,)),
            pltpu.SemaphoreType.REGULAR((N_DEV,)),
        ],
        compiler_params=pltpu.CompilerParams(
            vmem_limit_bytes=100 * 1024 * 1024,
            collective_id=0,
        ),
    )(x, w_mat)


# device time: 48319 ns/iter; 1.0518x vs baseline; 1.0433x over previous
import jax
import jax.numpy as jnp
from jax import lax
from jax.experimental import pallas as pl
from jax.experimental.pallas import tpu as pltpu

N_DEV = 16


def _gelu(y):
    c = 0.7978845608028654
    return 0.5 * y * (1.0 + jnp.tanh(c * (y + 0.044715 * y * y * y)))


def _peer_order():
    order = []
    for d in range(1, N_DEV // 2):
        order.extend([d, N_DEV - d])
    order.append(N_DEV // 2)
    return order


def kernel(x, w_mat):
    k_global, k_shard = x.shape
    n = w_mat.shape[1]
    m_per = k_global // N_DEV
    assert k_shard == m_per

    def body(x_ref, w_ref, out_ref, xsend_ref, xcomm_ref, send_sems, recv_sems):
        my = lax.axis_index("i")

        xsend_ref[:, :] = x_ref[:, :].astype(jnp.bfloat16)

        barrier_sem = pltpu.get_barrier_semaphore()
        for s in range(1, N_DEV):
            pl.semaphore_signal(
                barrier_sem, inc=1,
                device_id=(lax.rem(my + s, N_DEV),),
                device_id_type=pl.DeviceIdType.MESH,
            )
        acc = jnp.dot(
            x_ref[pl.ds(my * m_per, m_per), :],
            w_ref[pl.ds(my * m_per, m_per), :],
            preferred_element_type=jnp.float32,
        )
        pl.semaphore_wait(barrier_sem, N_DEV - 1)

        sends = []
        for idx, s in enumerate(_peer_order()):
            dest = lax.rem(my + s, N_DEV)
            rdma = pltpu.make_async_remote_copy(
                src_ref=xsend_ref.at[pl.ds(dest * m_per, m_per)],
                dst_ref=xcomm_ref.at[my],
                send_sem=send_sems.at[idx + 1],
                recv_sem=recv_sems.at[my],
                device_id=(dest,),
                device_id_type=pl.DeviceIdType.MESH,
            )
            rdma.start()
            sends.append(rdma)

        for s in _peer_order():
            src = lax.rem(my - s + N_DEV, N_DEV)
            recv = pltpu.make_async_remote_copy(
                src_ref=xsend_ref.at[pl.ds(0, m_per)],
                dst_ref=xcomm_ref.at[src],
                send_sem=send_sems.at[0],
                recv_sem=recv_sems.at[src],
                device_id=(src,),
                device_id_type=pl.DeviceIdType.MESH,
            )
            recv.wait_recv()
            acc = acc + jnp.dot(
                xcomm_ref[src].astype(jnp.float32),
                w_ref[pl.ds(src * m_per, m_per), :],
                preferred_element_type=jnp.float32,
            )

        out_ref[:, :] = _gelu(acc)

        for rdma in sends:
            rdma.wait_send()

    return pl.pallas_call(
        body,
        out_shape=jax.ShapeDtypeStruct((m_per, n), jnp.float32),
        in_specs=[
            pl.BlockSpec(memory_space=pltpu.VMEM),
            pl.BlockSpec(memory_space=pltpu.VMEM),
        ],
        out_specs=pl.BlockSpec(memory_space=pltpu.VMEM),
        scratch_shapes=[
            pltpu.VMEM((k_global, m_per), jnp.bfloat16),
            pltpu.VMEM((N_DEV, m_per, m_per), jnp.bfloat16),
            pltpu.SemaphoreType.DMA((N_DEV,)),
            pltpu.SemaphoreType.DMA((N_DEV,)),
        ],
        compiler_params=pltpu.CompilerParams(
            vmem_limit_bytes=100 * 1024 * 1024,
            collective_id=0,
        ),
    )(x, w_mat)
